# Initial kernel scaffold; baseline (speedup 1.0000x reference)
import jax
import jax.numpy as jnp
from jax.experimental import pallas as pl


def kernel(x, edge_index, W_g1, b_g1, W_g2, b_g2, W_g3, b_g3, Wp, bp, gamma, beta, W1, bc1, W2, bc2, Wl1, bl1, Wl2, bl2):
    raise NotImplementedError("write your pallas kernel here")


# speedup vs baseline: 7.3227x; 7.3227x over previous
import functools

import jax
import jax.numpy as jnp
from jax.experimental import pallas as pl
from jax.experimental.pallas import tpu as pltpu

N = 10000
B = 20
NPG = 500
PG = 512
DIMS = 250
PD = 256
NHID = 128
C1 = 32
C2 = 64
NCLS = 2
E = 320000

NEG_INF = float("-inf")
CK = 128


def _build_counts_xla(edge_index):
    src = edge_index[0]
    dst = edge_index[1]
    g = dst // NPG
    flat = g * (PG * PG) + (dst - g * NPG) * PG + (src - g * NPG)
    a = jnp.zeros((B * PG * PG,), jnp.float32)
    a = a.at[flat].add(1.0)
    return a.reshape(B, PG, PG)


def _matmul(p, q):
    return jax.lax.dot_general(p, q, (((1,), (0,)), ((), ())),
                               preferred_element_type=jnp.float32)


def _dense_kernel(a_ref, ni_ref, no_ref, x_ref, wg_ref, bg_ref, wp_ref,
                  bp_ref, gamma_ref, beta_ref, w1r_ref, bc1_ref, se_ref,
                  so_ref, q2_ref, bc2_ref, wl1_ref, bl1_ref, wl2_ref, bl2_ref,
                  out_ref):
    a = a_ref[0]
    norm_in = ni_ref[0, 0][:, None]
    norm_out = no_ref[0, 0][:, None]

    bp = bp_ref[0, 0]
    gamma = gamma_ref[0][None, :]
    beta = beta_ref[0][None, :]

    row_valid = jax.lax.broadcasted_iota(jnp.int32, (PG, 1), 0) < NPG
    ii_col = jax.lax.broadcasted_iota(jnp.int32, (PG, CK), 0)
    jj_loc = jax.lax.broadcasted_iota(jnp.int32, (PG, CK), 1)
    p_iota = jax.lax.broadcasted_iota(jnp.int32, (PD, PG), 0)

    pooled = []
    agg = _matmul(a, x_ref[0] * norm_out) * norm_in
    for layer in range(3):
        h = jnp.maximum(_matmul(agg, wg_ref[layer]) + bg_ref[layer][None, :],
                        0.0)
        agg = _matmul(a, h * norm_out) * norm_in
        score = jnp.sum(_matmul(agg, wp_ref[...]), axis=1,
                        keepdims=True) + bp
        score = jnp.where(row_valid, score, NEG_INF)
        s_row = jnp.transpose(score, (1, 0))
        si = jnp.broadcast_to(score, (PG, CK))

        rank = jnp.zeros((PG, 1), jnp.int32)
        for jc in range(PG // CK):
            sj = jnp.broadcast_to(s_row[:, jc * CK:(jc + 1) * CK], (PG, CK))
            jj = jj_loc + jc * CK
            cmp = (sj > si) | ((sj == si) & (jj < ii_col))
            rank = rank + jnp.sum(cmp.astype(jnp.int32), axis=1,
                                  keepdims=True)
        rank_row = jnp.transpose(rank, (1, 0))
        sel = (p_iota == rank_row).astype(jnp.float32)
        topv = jnp.sum(sel * jnp.broadcast_to(s_row, (PD, PG)), axis=1,
                       keepdims=True)
        xp = _matmul(sel, h) * jnp.tanh(topv)
        mu = jnp.mean(xp, axis=1, keepdims=True)
        var = jnp.mean((xp - mu) * (xp - mu), axis=1, keepdims=True)
        ln = (xp - mu) * jax.lax.rsqrt(var + 1e-5)
        pooled.append(ln * gamma + beta)

    xc = jnp.concatenate(pooled, axis=0)
    y1 = jnp.maximum(_matmul(w1r_ref[...], xc) + bc1_ref[...], 0.0)
    yp = jnp.maximum(_matmul(y1, se_ref[...]), _matmul(y1, so_ref[...]))
    z = jnp.zeros((1, C2), jnp.float32)
    for c in range(C1):
        z = z + _matmul(yp[c:c + 1, :], q2_ref[c])
    z = jnp.maximum(z + bc2_ref[...], 0.0)
    l1 = jnp.maximum(_matmul(z, wl1_ref[...]) + bl1_ref[...], 0.0)
    lg = _matmul(l1, wl2_ref[...]) + bl2_ref[...]
    col = jax.lax.broadcasted_iota(jnp.int32, (1, NHID), 1)
    lgm = jnp.where(col < NCLS, lg, NEG_INF)
    m = jnp.max(lgm, axis=1, keepdims=True)
    den = jnp.sum(jnp.where(col < NCLS, jnp.exp(lgm - m), 0.0), axis=1,
                  keepdims=True)
    out_ref[0] = lgm - m - jnp.log(den)


def _const_spec(shape):
    nd = len(shape)
    return pl.BlockSpec(shape, lambda g: (0,) * nd)


def _dense_phase(counts, x, W_g1, b_g1, W_g2, b_g2, W_g3, b_g3, Wp, bp, gamma,
                 beta, W1, bc1, W2, bc2, Wl1, bl1, Wl2, bl2):
    deg_in = jnp.sum(counts, axis=2)
    deg_out = jnp.sum(counts, axis=1)
    ni = jnp.where(deg_in > 0, deg_in ** -0.5, 0.0)
    no = jnp.where(deg_out > 0, deg_out ** -0.5, 0.0)

    xp3 = jnp.pad(x.reshape(B, NPG, NHID), ((0, 0), (0, PG - NPG), (0, 0)))
    wg = jnp.stack([W_g1, W_g2, W_g3])
    bg = jnp.stack([b_g1, b_g2, b_g3])
    wp_pad = jnp.pad(Wp, ((0, 0), (0, NHID - 1)))
    bp2 = bp.reshape(1, 1)
    gamma2 = gamma[None, :]
    beta2 = beta[None, :]
    w1r = jnp.pad(W1[:, :, :, 0], ((0, 0), (0, 0), (0, PD - DIMS))) \
        .reshape(C1, 3 * PD)
    bc1b = jnp.broadcast_to(bc1[:, None], (C1, NHID))
    eye = jnp.eye(NHID, dtype=jnp.float32)
    se_m = eye[:, 0::2]
    so_m = eye[:, 1::2]
    q2 = jnp.transpose(W2[:, :, 0, :], (1, 2, 0))
    bc2r = bc2[None, :]
    bl1r = bl1[None, :]
    wl2p = jnp.pad(Wl2, ((0, 0), (0, NHID - NCLS)))
    bl2p = jnp.pad(bl2, (0, NHID - NCLS))[None, :]

    out = pl.pallas_call(
        _dense_kernel,
        grid=(B,),
        in_specs=[
            pl.BlockSpec((1, PG, PG), lambda g: (g, 0, 0)),
            pl.BlockSpec((1, 1, PG), lambda g: (g, 0, 0)),
            pl.BlockSpec((1, 1, PG), lambda g: (g, 0, 0)),
            pl.BlockSpec((1, PG, NHID), lambda g: (g, 0, 0)),
            _const_spec((3, NHID, NHID)),
            _const_spec((3, NHID)),
            _const_spec((NHID, NHID)),
            _const_spec((1, 1)),
            _const_spec((1, NHID)),
            _const_spec((1, NHID)),
            _const_spec((C1, 3 * PD)),
            _const_spec((C1, NHID)),
            _const_spec((NHID, C2)),
            _const_spec((NHID, C2)),
            _const_spec((C1, C2, C2)),
            _const_spec((1, C2)),
            _const_spec((C2, C2)),
            _const_spec((1, C2)),
            _const_spec((C2, NHID)),
            _const_spec((1, NHID)),
        ],
        out_specs=pl.BlockSpec((1, 1, NHID), lambda g: (g, 0, 0)),
        out_shape=jax.ShapeDtypeStruct((B, 1, NHID), jnp.float32),
        compiler_params=pltpu.CompilerParams(
            dimension_semantics=("parallel",)),
    )(counts, ni[:, None, :], no[:, None, :], xp3, wg, bg, wp_pad, bp2, gamma2, beta2, w1r, bc1b,
      se_m, so_m, q2, bc2r, Wl1, bl1r, wl2p, bl2p)
    return out[:, 0, :NCLS]


@jax.jit
def kernel(x, edge_index, W_g1, b_g1, W_g2, b_g2, W_g3, b_g3, Wp, bp, gamma,
           beta, W1, bc1, W2, bc2, Wl1, bl1, Wl2, bl2):
    counts = _build_counts_xla(edge_index)
    return _dense_phase(counts, x, W_g1, b_g1, W_g2, b_g2, W_g3, b_g3, Wp, bp,
                        gamma, beta, W1, bc1, W2, bc2, Wl1, bl1, Wl2, bl2)


# speedup vs baseline: 25.2672x; 3.4505x over previous
import functools

import jax
import jax.numpy as jnp
from jax.experimental import pallas as pl
from jax.experimental.pallas import tpu as pltpu

N = 10000
B = 20
NPG = 500
PG = 512
DIMS = 250
PD = 256
NHID = 128
C1 = 32
C2 = 64
NCLS = 2
E = 320000

NEG_INF = float("-inf")
CK = 128


def _build_counts_xla(edge_index):
    src = edge_index[0]
    dst = edge_index[1]
    g = dst // NPG
    flat = g * (PG * PG) + (dst - g * NPG) * PG + (src - g * NPG)
    a = jnp.zeros((B * PG * PG,), jnp.float32)
    a = a.at[flat].add(1.0)
    return a.reshape(B, PG, PG)


NSC = 2
NTL = 16
EPT = E // NTL
CH = 2000
NROW = 16
ACH = 5 * PG * PG
TRASH = ACH
ZW = 8192


def _sc_body(src_hbm, dst_hbm, out_hbm, src_v, dst_v, idx_row, val_row, zbuf,
             a_sh):
    from jax.experimental.pallas import tpu_sc as plsc
    cid = jax.lax.axis_index("c")
    sid = jax.lax.axis_index("s")
    lane = jax.lax.iota(jnp.int32, 16)
    trash_base = TRASH + sid * 16 + lane

    def fill16(i, ref, value):
        ref[pl.ds(i * 16, 16)] = jnp.full((16,), value, ref.dtype)
        return 0

    jax.lax.fori_loop(0, ZW // 16, lambda i, _: fill16(i, zbuf, 0.0), 0)
    for t in range(8):
        fill16(t, val_row, 1.0)

    for p in range(2):
        chunk = p * NSC + cid
        chunk_lo = chunk * 5
        for k in range(ACH // (NTL * ZW)):
            pltpu.sync_copy(zbuf, a_sh.at[pl.ds(sid * (ACH // NTL) + k * ZW,
                                                ZW)])
        plsc.subcore_barrier()

        def buffer_step(bi, _):
            off = sid * EPT + bi * CH
            pltpu.sync_copy(src_hbm.at[pl.ds(off, CH)], src_v)
            pltpu.sync_copy(dst_hbm.at[pl.ds(off, CH)], dst_v)

            def row_step(r, _):
                for w in range(8):
                    _edge_vec(r * 128 + w * 16, w * 16)
                pltpu.sync_copy(val_row, a_sh.at[idx_row], add=True)
                return 0

            def _edge_vec(start, local):
                sv = src_v[pl.ds(start, 16)]
                dv = dst_v[pl.ds(start, 16)]
                g = jax.lax.shift_right_logical(dv * 16778, 23)
                lg = g - chunk_lo
                ok = (lg >= 0) & (lg < 5)
                flat = (lg * (PG * PG) + (dv - g * NPG) * PG + (sv - g * NPG))
                idx_row[pl.ds(local, 16)] = jnp.where(ok, flat, trash_base)

            jax.lax.fori_loop(0, CH // 128, row_step, 0)
            base = (CH // 128) * 128
            for w in range(5):
                _edge_vec(base + w * 16, w * 16)
            for t in range(5, 8):
                idx_row[pl.ds(t * 16, 16)] = trash_base
            pltpu.sync_copy(val_row, a_sh.at[idx_row], add=True)
            return 0

        jax.lax.fori_loop(0, EPT // CH, buffer_step, 0)
        plsc.subcore_barrier()
        pltpu.sync_copy(
            a_sh.at[pl.ds(sid * (ACH // NTL), ACH // NTL)],
            out_hbm.at[pl.ds(chunk * ACH + sid * (ACH // NTL), ACH // NTL)])
        plsc.subcore_barrier()


def _build_counts_sc(edge_index):
    from jax.experimental.pallas import tpu_sc as plsc
    mesh = plsc.VectorSubcoreMesh(core_axis_name="c", subcore_axis_name="s")
    flat = pl.kernel(
        _sc_body,
        mesh=mesh,
        out_type=jax.ShapeDtypeStruct((B * PG * PG,), jnp.float32),
        scratch_types=[
            pltpu.VMEM((CH,), jnp.int32),
            pltpu.VMEM((CH,), jnp.int32),
            pltpu.VMEM((128,), jnp.int32),
            pltpu.VMEM((128,), jnp.float32),
            pltpu.VMEM((ZW,), jnp.float32),
            pltpu.VMEM_SHARED((ACH + 512,), jnp.float32),
        ],
    )(edge_index[0], edge_index[1])
    return flat.reshape(B, PG, PG)


def _matmul(p, q):
    return jax.lax.dot_general(p, q, (((1,), (0,)), ((), ())),
                               preferred_element_type=jnp.float32)


def _dense_kernel(a_ref, ni_ref, no_ref, x_ref, wg_ref, bg_ref, wp_ref,
                  bp_ref, gamma_ref, beta_ref, w1r_ref, bc1_ref, se_ref,
                  so_ref, q2_ref, bc2_ref, wl1_ref, bl1_ref, wl2_ref, bl2_ref,
                  out_ref):
    a = a_ref[0]
    norm_in = ni_ref[0, 0][:, None]
    norm_out = no_ref[0, 0][:, None]

    bp = bp_ref[0, 0]
    gamma = gamma_ref[0][None, :]
    beta = beta_ref[0][None, :]

    row_valid = jax.lax.broadcasted_iota(jnp.int32, (PG, 1), 0) < NPG
    ii_col = jax.lax.broadcasted_iota(jnp.int32, (PG, CK), 0)
    jj_loc = jax.lax.broadcasted_iota(jnp.int32, (PG, CK), 1)
    p_iota = jax.lax.broadcasted_iota(jnp.int32, (PD, PG), 0)

    pooled = []
    agg = _matmul(a, x_ref[0] * norm_out) * norm_in
    for layer in range(3):
        h = jnp.maximum(_matmul(agg, wg_ref[layer]) + bg_ref[layer][None, :],
                        0.0)
        agg = _matmul(a, h * norm_out) * norm_in
        score = jnp.sum(_matmul(agg, wp_ref[...]), axis=1,
                        keepdims=True) + bp
        score = jnp.where(row_valid, score, NEG_INF)
        s_row = jnp.transpose(score, (1, 0))
        si = jnp.broadcast_to(score, (PG, CK))

        rank = jnp.zeros((PG, 1), jnp.int32)
        for jc in range(PG // CK):
            sj = jnp.broadcast_to(s_row[:, jc * CK:(jc + 1) * CK], (PG, CK))
            jj = jj_loc + jc * CK
            cmp = (sj > si) | ((sj == si) & (jj < ii_col))
            rank = rank + jnp.sum(cmp.astype(jnp.int32), axis=1,
                                  keepdims=True)
        rank_row = jnp.transpose(rank, (1, 0))
        sel = (p_iota == rank_row).astype(jnp.float32)
        topv = jnp.sum(sel * jnp.broadcast_to(s_row, (PD, PG)), axis=1,
                       keepdims=True)
        xp = _matmul(sel, h) * jnp.tanh(topv)
        mu = jnp.mean(xp, axis=1, keepdims=True)
        var = jnp.mean((xp - mu) * (xp - mu), axis=1, keepdims=True)
        ln = (xp - mu) * jax.lax.rsqrt(var + 1e-5)
        pooled.append(ln * gamma + beta)

    xc = jnp.concatenate(pooled, axis=0)
    y1 = jnp.maximum(_matmul(w1r_ref[...], xc) + bc1_ref[...], 0.0)
    yp = jnp.maximum(_matmul(y1, se_ref[...]), _matmul(y1, so_ref[...]))
    z = jnp.zeros((1, C2), jnp.float32)
    for c in range(C1):
        z = z + _matmul(yp[c:c + 1, :], q2_ref[c])
    z = jnp.maximum(z + bc2_ref[...], 0.0)
    l1 = jnp.maximum(_matmul(z, wl1_ref[...]) + bl1_ref[...], 0.0)
    lg = _matmul(l1, wl2_ref[...]) + bl2_ref[...]
    col = jax.lax.broadcasted_iota(jnp.int32, (1, NHID), 1)
    lgm = jnp.where(col < NCLS, lg, NEG_INF)
    m = jnp.max(lgm, axis=1, keepdims=True)
    den = jnp.sum(jnp.where(col < NCLS, jnp.exp(lgm - m), 0.0), axis=1,
                  keepdims=True)
    out_ref[0] = lgm - m - jnp.log(den)


def _const_spec(shape):
    nd = len(shape)
    return pl.BlockSpec(shape, lambda g: (0,) * nd)


def _dense_phase(counts, x, W_g1, b_g1, W_g2, b_g2, W_g3, b_g3, Wp, bp, gamma,
                 beta, W1, bc1, W2, bc2, Wl1, bl1, Wl2, bl2):
    deg_in = jnp.sum(counts, axis=2)
    deg_out = jnp.sum(counts, axis=1)
    ni = jnp.where(deg_in > 0, deg_in ** -0.5, 0.0)
    no = jnp.where(deg_out > 0, deg_out ** -0.5, 0.0)

    xp3 = jnp.pad(x.reshape(B, NPG, NHID), ((0, 0), (0, PG - NPG), (0, 0)))
    wg = jnp.stack([W_g1, W_g2, W_g3])
    bg = jnp.stack([b_g1, b_g2, b_g3])
    wp_pad = jnp.pad(Wp, ((0, 0), (0, NHID - 1)))
    bp2 = bp.reshape(1, 1)
    gamma2 = gamma[None, :]
    beta2 = beta[None, :]
    w1r = jnp.pad(W1[:, :, :, 0], ((0, 0), (0, 0), (0, PD - DIMS))) \
        .reshape(C1, 3 * PD)
    bc1b = jnp.broadcast_to(bc1[:, None], (C1, NHID))
    eye = jnp.eye(NHID, dtype=jnp.float32)
    se_m = eye[:, 0::2]
    so_m = eye[:, 1::2]
    q2 = jnp.transpose(W2[:, :, 0, :], (1, 2, 0))
    bc2r = bc2[None, :]
    bl1r = bl1[None, :]
    wl2p = jnp.pad(Wl2, ((0, 0), (0, NHID - NCLS)))
    bl2p = jnp.pad(bl2, (0, NHID - NCLS))[None, :]

    out = pl.pallas_call(
        _dense_kernel,
        grid=(B,),
        in_specs=[
            pl.BlockSpec((1, PG, PG), lambda g: (g, 0, 0)),
            pl.BlockSpec((1, 1, PG), lambda g: (g, 0, 0)),
            pl.BlockSpec((1, 1, PG), lambda g: (g, 0, 0)),
            pl.BlockSpec((1, PG, NHID), lambda g: (g, 0, 0)),
            _const_spec((3, NHID, NHID)),
            _const_spec((3, NHID)),
            _const_spec((NHID, NHID)),
            _const_spec((1, 1)),
            _const_spec((1, NHID)),
            _const_spec((1, NHID)),
            _const_spec((C1, 3 * PD)),
            _const_spec((C1, NHID)),
            _const_spec((NHID, C2)),
            _const_spec((NHID, C2)),
            _const_spec((C1, C2, C2)),
            _const_spec((1, C2)),
            _const_spec((C2, C2)),
            _const_spec((1, C2)),
            _const_spec((C2, NHID)),
            _const_spec((1, NHID)),
        ],
        out_specs=pl.BlockSpec((1, 1, NHID), lambda g: (g, 0, 0)),
        out_shape=jax.ShapeDtypeStruct((B, 1, NHID), jnp.float32),
        compiler_params=pltpu.CompilerParams(
            dimension_semantics=("parallel",)),
    )(counts, ni[:, None, :], no[:, None, :], xp3, wg, bg, wp_pad, bp2, gamma2, beta2, w1r, bc1b,
      se_m, so_m, q2, bc2r, Wl1, bl1r, wl2p, bl2p)
    return out[:, 0, :NCLS]


@jax.jit
def kernel(x, edge_index, W_g1, b_g1, W_g2, b_g2, W_g3, b_g3, Wp, bp, gamma,
           beta, W1, bc1, W2, bc2, Wl1, bl1, Wl2, bl2):
    counts = _build_counts_sc(edge_index)
    return _dense_phase(counts, x, W_g1, b_g1, W_g2, b_g2, W_g3, b_g3, Wp, bp,
                        gamma, beta, W1, bc1, W2, bc2, Wl1, bl1, Wl2, bl2)


# speedup vs baseline: 29.3159x; 1.1602x over previous
import functools

import jax
import jax.numpy as jnp
from jax.experimental import pallas as pl
from jax.experimental.pallas import tpu as pltpu

N = 10000
B = 20
NPG = 500
PG = 512
DIMS = 250
PD = 256
NHID = 128
C1 = 32
C2 = 64
NCLS = 2
E = 320000

NEG_INF = float("-inf")
CK = 128


def _build_counts_xla(edge_index):
    src = edge_index[0]
    dst = edge_index[1]
    g = dst // NPG
    flat = g * (PG * PG) + (dst - g * NPG) * PG + (src - g * NPG)
    a = jnp.zeros((B * PG * PG,), jnp.float32)
    a = a.at[flat].add(1.0)
    return a.reshape(B, PG, PG)


NSC = 2
NTL = 16
EPT = E // NTL
CH = 2000
NROW = 16
ACH = 5 * PG * PG
TRASH = ACH
ZW = 8192


def _sc_body(src_hbm, dst_hbm, out_hbm, src_v, dst_v, idx_buf, val_row, zbuf,
             a_sh, sem):
    from jax.experimental.pallas import tpu_sc as plsc
    cid = jax.lax.axis_index("c")
    sid = jax.lax.axis_index("s")
    lane = jax.lax.iota(jnp.int32, 16)
    trash_base = TRASH + sid * 16 + lane

    def fill16(i, ref, value):
        ref[pl.ds(i * 16, 16)] = jnp.full((16,), value, ref.dtype)
        return 0

    jax.lax.fori_loop(0, ZW // 16, lambda i, _: fill16(i, zbuf, 0.0), 0)
    for t in range(8):
        fill16(t, val_row, 1.0)

    for p in range(2):
        chunk = p * NSC + cid
        chunk_lo = chunk * 5
        zcopies = [
            pltpu.make_async_copy(
                zbuf, a_sh.at[pl.ds(sid * (ACH // NTL) + k * ZW, ZW)], sem)
            for k in range(ACH // (NTL * ZW))
        ]
        for c in zcopies:
            c.start()
        for c in zcopies:
            c.wait()
        plsc.subcore_barrier()

        def buffer_step(bi, _):
            off = sid * EPT + bi * CH
            pltpu.sync_copy(src_hbm.at[pl.ds(off, CH)], src_v)
            pltpu.sync_copy(dst_hbm.at[pl.ds(off, CH)], dst_v)

            def _edge_vec(start, r, w):
                sv = src_v[pl.ds(start, 16)]
                dv = dst_v[pl.ds(start, 16)]
                g = jax.lax.shift_right_logical(dv * 16778, 23)
                lg = g - chunk_lo
                ok = (lg >= 0) & (lg < 5)
                flat = (lg * (PG * PG) + (dv - g * NPG) * PG + (sv - g * NPG))
                idx_buf[r, pl.ds(w * 16, 16)] = jnp.where(ok, flat,
                                                          trash_base)

            copies = []
            for r in range(16):
                nw = 8 if r < 15 else 5
                for w in range(nw):
                    _edge_vec(r * 128 + w * 16, r, w)
                if r == 15:
                    for t in range(5, 8):
                        idx_buf[15, pl.ds(t * 16, 16)] = trash_base
                copies.append(pltpu.async_copy(
                    val_row, a_sh.at[idx_buf.at[r]], sem, add=True))
            for c in copies:
                c.wait()
            return 0

        jax.lax.fori_loop(0, EPT // CH, buffer_step, 0)
        plsc.subcore_barrier()
        pltpu.sync_copy(
            a_sh.at[pl.ds(sid * (ACH // NTL), ACH // NTL)],
            out_hbm.at[pl.ds(chunk * ACH + sid * (ACH // NTL), ACH // NTL)])
        plsc.subcore_barrier()


def _build_counts_sc(edge_index):
    from jax.experimental.pallas import tpu_sc as plsc
    mesh = plsc.VectorSubcoreMesh(core_axis_name="c", subcore_axis_name="s")
    flat = pl.kernel(
        _sc_body,
        mesh=mesh,
        out_type=jax.ShapeDtypeStruct((B * PG * PG,), jnp.float32),
        scratch_types=[
            pltpu.VMEM((CH,), jnp.int32),
            pltpu.VMEM((CH,), jnp.int32),
            pltpu.VMEM((16, 128), jnp.int32),
            pltpu.VMEM((128,), jnp.float32),
            pltpu.VMEM((ZW,), jnp.float32),
            pltpu.VMEM_SHARED((ACH + 512,), jnp.float32),
            pltpu.SemaphoreType.DMA,
        ],
    )(edge_index[0], edge_index[1])
    return flat.reshape(B, PG, PG)


def _matmul(p, q):
    return jax.lax.dot_general(p, q, (((1,), (0,)), ((), ())),
                               preferred_element_type=jnp.float32)


def _dense_kernel(a_ref, ni_ref, no_ref, x_ref, wg_ref, bg_ref, wp_ref,
                  bp_ref, gamma_ref, beta_ref, w1r_ref, bc1_ref, se_ref,
                  so_ref, q2_ref, bc2_ref, wl1_ref, bl1_ref, wl2_ref, bl2_ref,
                  out_ref):
    a = a_ref[0]
    norm_in = ni_ref[0, 0][:, None]
    norm_out = no_ref[0, 0][:, None]

    bp = bp_ref[0, 0]
    gamma = gamma_ref[0][None, :]
    beta = beta_ref[0][None, :]

    row_valid = jax.lax.broadcasted_iota(jnp.int32, (PG, 1), 0) < NPG
    ii_col = jax.lax.broadcasted_iota(jnp.int32, (PG, CK), 0)
    jj_loc = jax.lax.broadcasted_iota(jnp.int32, (PG, CK), 1)
    p_iota = jax.lax.broadcasted_iota(jnp.int32, (PD, PG), 0)

    pooled = []
    agg = _matmul(a, x_ref[0] * norm_out) * norm_in
    for layer in range(3):
        h = jnp.maximum(_matmul(agg, wg_ref[layer]) + bg_ref[layer][None, :],
                        0.0)
        agg = _matmul(a, h * norm_out) * norm_in
        score = jnp.sum(_matmul(agg, wp_ref[...]), axis=1,
                        keepdims=True) + bp
        score = jnp.where(row_valid, score, NEG_INF)
        s_row = jnp.transpose(score, (1, 0))
        si = jnp.broadcast_to(score, (PG, CK))

        rank = jnp.zeros((PG, 1), jnp.int32)
        for jc in range(PG // CK):
            sj = jnp.broadcast_to(s_row[:, jc * CK:(jc + 1) * CK], (PG, CK))
            jj = jj_loc + jc * CK
            cmp = (sj > si) | ((sj == si) & (jj < ii_col))
            rank = rank + jnp.sum(cmp.astype(jnp.int32), axis=1,
                                  keepdims=True)
        rank_row = jnp.transpose(rank, (1, 0))
        sel = (p_iota == rank_row).astype(jnp.float32)
        topv = jnp.sum(sel * jnp.broadcast_to(s_row, (PD, PG)), axis=1,
                       keepdims=True)
        xp = _matmul(sel, h) * jnp.tanh(topv)
        mu = jnp.mean(xp, axis=1, keepdims=True)
        var = jnp.mean((xp - mu) * (xp - mu), axis=1, keepdims=True)
        ln = (xp - mu) * jax.lax.rsqrt(var + 1e-5)
        pooled.append(ln * gamma + beta)

    xc = jnp.concatenate(pooled, axis=0)
    y1 = jnp.maximum(_matmul(w1r_ref[...], xc) + bc1_ref[...], 0.0)
    yp = jnp.maximum(_matmul(y1, se_ref[...]), _matmul(y1, so_ref[...]))
    z = jnp.zeros((1, C2), jnp.float32)
    for c in range(C1):
        z = z + _matmul(yp[c:c + 1, :], q2_ref[c])
    z = jnp.maximum(z + bc2_ref[...], 0.0)
    l1 = jnp.maximum(_matmul(z, wl1_ref[...]) + bl1_ref[...], 0.0)
    lg = _matmul(l1, wl2_ref[...]) + bl2_ref[...]
    col = jax.lax.broadcasted_iota(jnp.int32, (1, NHID), 1)
    lgm = jnp.where(col < NCLS, lg, NEG_INF)
    m = jnp.max(lgm, axis=1, keepdims=True)
    den = jnp.sum(jnp.where(col < NCLS, jnp.exp(lgm - m), 0.0), axis=1,
                  keepdims=True)
    out_ref[0] = lgm - m - jnp.log(den)


def _const_spec(shape):
    nd = len(shape)
    return pl.BlockSpec(shape, lambda g: (0,) * nd)


def _dense_phase(counts, x, W_g1, b_g1, W_g2, b_g2, W_g3, b_g3, Wp, bp, gamma,
                 beta, W1, bc1, W2, bc2, Wl1, bl1, Wl2, bl2):
    deg_in = jnp.sum(counts, axis=2)
    deg_out = jnp.sum(counts, axis=1)
    ni = jnp.where(deg_in > 0, deg_in ** -0.5, 0.0)
    no = jnp.where(deg_out > 0, deg_out ** -0.5, 0.0)

    xp3 = jnp.pad(x.reshape(B, NPG, NHID), ((0, 0), (0, PG - NPG), (0, 0)))
    wg = jnp.stack([W_g1, W_g2, W_g3])
    bg = jnp.stack([b_g1, b_g2, b_g3])
    wp_pad = jnp.pad(Wp, ((0, 0), (0, NHID - 1)))
    bp2 = bp.reshape(1, 1)
    gamma2 = gamma[None, :]
    beta2 = beta[None, :]
    w1r = jnp.pad(W1[:, :, :, 0], ((0, 0), (0, 0), (0, PD - DIMS))) \
        .reshape(C1, 3 * PD)
    bc1b = jnp.broadcast_to(bc1[:, None], (C1, NHID))
    eye = jnp.eye(NHID, dtype=jnp.float32)
    se_m = eye[:, 0::2]
    so_m = eye[:, 1::2]
    q2 = jnp.transpose(W2[:, :, 0, :], (1, 2, 0))
    bc2r = bc2[None, :]
    bl1r = bl1[None, :]
    wl2p = jnp.pad(Wl2, ((0, 0), (0, NHID - NCLS)))
    bl2p = jnp.pad(bl2, (0, NHID - NCLS))[None, :]

    out = pl.pallas_call(
        _dense_kernel,
        grid=(B,),
        in_specs=[
            pl.BlockSpec((1, PG, PG), lambda g: (g, 0, 0)),
            pl.BlockSpec((1, 1, PG), lambda g: (g, 0, 0)),
            pl.BlockSpec((1, 1, PG), lambda g: (g, 0, 0)),
            pl.BlockSpec((1, PG, NHID), lambda g: (g, 0, 0)),
            _const_spec((3, NHID, NHID)),
            _const_spec((3, NHID)),
            _const_spec((NHID, NHID)),
            _const_spec((1, 1)),
            _const_spec((1, NHID)),
            _const_spec((1, NHID)),
            _const_spec((C1, 3 * PD)),
            _const_spec((C1, NHID)),
            _const_spec((NHID, C2)),
            _const_spec((NHID, C2)),
            _const_spec((C1, C2, C2)),
            _const_spec((1, C2)),
            _const_spec((C2, C2)),
            _const_spec((1, C2)),
            _const_spec((C2, NHID)),
            _const_spec((1, NHID)),
        ],
        out_specs=pl.BlockSpec((1, 1, NHID), lambda g: (g, 0, 0)),
        out_shape=jax.ShapeDtypeStruct((B, 1, NHID), jnp.float32),
        compiler_params=pltpu.CompilerParams(
            dimension_semantics=("parallel",)),
    )(counts, ni[:, None, :], no[:, None, :], xp3, wg, bg, wp_pad, bp2, gamma2, beta2, w1r, bc1b,
      se_m, so_m, q2, bc2r, Wl1, bl1r, wl2p, bl2p)
    return out[:, 0, :NCLS]


@jax.jit
def kernel(x, edge_index, W_g1, b_g1, W_g2, b_g2, W_g3, b_g3, Wp, bp, gamma,
           beta, W1, bc1, W2, bc2, Wl1, bl1, Wl2, bl2):
    counts = _build_counts_sc(edge_index)
    return _dense_phase(counts, x, W_g1, b_g1, W_g2, b_g2, W_g3, b_g3, Wp, bp,
                        gamma, beta, W1, bc1, W2, bc2, Wl1, bl1, Wl2, bl2)
